# baseline (speedup 1.0000x reference)
import functools

import jax
import jax.numpy as jnp
from jax import lax
from jax.experimental import pallas as pl
from jax.experimental.pallas import tpu as pltpu
from jax.experimental.pallas import tpu_sc as plsc

N = 16384
T = 200
B = N * T
D = 32
NC, NS = 2, 16
NW = NC * NS
BPW = B // NW
CN = 512
NCH = BPW // CN
NRB = 4
NGRP = NCH // NRB
CB = CN // 128
L = 16
P = 144

_mesh = plsc.VectorSubcoreMesh(core_axis_name="c", subcore_axis_name="s")


@functools.partial(
    pl.kernel,
    out_type=jax.ShapeDtypeStruct((T, D // 8, N // 128, 8, 128), jnp.float32),
    mesh=_mesh,
    scratch_types=[
        pltpu.VMEM((NRB, CN), jnp.int32),
        pltpu.VMEM((NRB, CN, D), jnp.float32),
        pltpu.VMEM((2, D // 8, CB, 8, P), jnp.float32),
        [pltpu.SemaphoreType.DMA] * NRB,
        [pltpu.SemaphoreType.DMA] * NRB,
        [pltpu.SemaphoreType.DMA] * 2,
    ],
    compiler_params=pltpu.CompilerParams(
        use_tc_tiling_on_sc=False, needs_layout_passes=False
    ),
)
def _gather_t(idx_hbm, table_hbm, out_hbm, idx_v, rows_v, tile_v,
              isems, gsems, wsems):
    wid = lax.axis_index("s") * NC + lax.axis_index("c")
    base = wid * BPW
    lane = lax.iota(jnp.int32, L)
    rb0 = lax.shift_right_logical(lane, 3)
    rb1 = rb0 + 2
    rr = lane & 7

    def coords(k):
        g0 = base + k * CN
        t = g0 >> 14
        cb0 = pl.multiple_of((g0 & (N - 1)) >> 7, CB)
        return g0, t, cb0

    def transpose_chunk(b, tb):
        tp = tile_v.at[tb]

        @plsc.parallel_loop(0, CN, step=8, unroll=4)
        def nblk(n0b):
            n0b = pl.multiple_of(n0b, 8)
            m = lax.shift_right_logical(n0b, 7)
            mv = jnp.full((L,), 0, jnp.int32) + m
            j0 = n0b & 127
            for u in range(8):
                n = n0b + u
                jv = jnp.full((L,), 0, jnp.int32) + (j0 + u)
                v0 = rows_v[b, n, pl.ds(0, L)]
                v1 = rows_v[b, n, pl.ds(L, L)]
                plsc.store_scatter(tp, [rb0, mv, rr, jv], v0)
                plsc.store_scatter(tp, [rb1, mv, rr, jv], v1)

    for b in range(NRB):
        g0, _, _ = coords(b)
        pltpu.sync_copy(idx_hbm.at[pl.ds(g0, CN)], idx_v.at[b])
        pltpu.async_copy(table_hbm.at[idx_v.at[b]], rows_v.at[b], gsems[b])

    def group(g, carry):
        for b in range(NRB):
            k = NRB * g + b
            tb = b % 2
            g0, t, cb0 = coords(k)
            pltpu.make_async_copy(
                table_hbm.at[idx_v.at[b]], rows_v.at[b], gsems[b]
            ).wait()

            @pl.when(g < NGRP - 1)
            def _pfi(b=b, k=k):
                g0n, _, _ = coords(k + NRB)
                pltpu.async_copy(
                    idx_hbm.at[pl.ds(g0n, CN)], idx_v.at[b], isems[b]
                )

            @pl.when(k >= 2)
            def _drain(tb=tb, k=k):
                _, tpp, cb0p = coords(k - 2)
                pltpu.make_async_copy(
                    tile_v.at[tb, :, :, :, pl.ds(0, 128)],
                    out_hbm.at[tpp, :, pl.ds(cb0p, CB)],
                    wsems[tb],
                ).wait()

            transpose_chunk(b, tb)
            pltpu.async_copy(
                tile_v.at[tb, :, :, :, pl.ds(0, 128)],
                out_hbm.at[t, :, pl.ds(cb0, CB)],
                wsems[tb],
            )

            @pl.when(g < NGRP - 1)
            def _pfg(b=b):
                pltpu.make_async_copy(
                    idx_hbm.at[pl.ds(0, CN)], idx_v.at[b], isems[b]
                ).wait()
                pltpu.async_copy(
                    table_hbm.at[idx_v.at[b]], rows_v.at[b], gsems[b]
                )

        return carry

    lax.fori_loop(0, NGRP, group, 0)
    for b in range(2):
        k = NCH - 2 + b
        _, t, cb0 = coords(k)
        pltpu.make_async_copy(
            tile_v.at[k % 2, :, :, :, pl.ds(0, 128)],
            out_hbm.at[t, :, pl.ds(cb0, CB)],
            wsems[k % 2],
        ).wait()


def kernel(round_numbers, embedding_table):
    idx_t = round_numbers.T.reshape(-1)
    out5 = _gather_t(idx_t, embedding_table)
    out_t = out5.transpose(0, 1, 3, 2, 4).reshape(T, D, N)
    return jnp.transpose(out_t, (2, 0, 1))


# speedup vs baseline: 1.0771x; 1.0771x over previous
import functools

import jax
import jax.numpy as jnp
from jax import lax
from jax.experimental import pallas as pl
from jax.experimental.pallas import tpu as pltpu
from jax.experimental.pallas import tpu_sc as plsc

N = 16384
T = 200
B = N * T
D = 32
NC, NS = 2, 16
NW = NC * NS
BPW = B // NW
CN = 512
NCH = BPW // CN
NRB = 4
NGRP = NCH // NRB
CB = CN // 128
L = 16
P = 144

_mesh = plsc.VectorSubcoreMesh(core_axis_name="c", subcore_axis_name="s")


@functools.partial(
    pl.kernel,
    out_type=jax.ShapeDtypeStruct((T, D // 8, N // 128, 8, 128), jnp.float32),
    mesh=_mesh,
    scratch_types=[
        pltpu.VMEM((NRB, CN), jnp.int32),
        pltpu.VMEM((NRB, CN, D), jnp.float32),
        pltpu.VMEM((2, D // 8, CB, 8, P), jnp.float32),
        [pltpu.SemaphoreType.DMA] * NRB,
        [pltpu.SemaphoreType.DMA] * NRB,
        [pltpu.SemaphoreType.DMA] * 2,
    ],
    compiler_params=pltpu.CompilerParams(
        use_tc_tiling_on_sc=False, needs_layout_passes=False
    ),
)
def _gather_t(idx_hbm, table_hbm, out_hbm, idx_v, rows_v, tile_v,
              isems, gsems, wsems):
    wid = lax.axis_index("s") * NC + lax.axis_index("c")
    base = wid * BPW
    lane = lax.iota(jnp.int32, L)
    rb0 = lax.shift_right_logical(lane, 3)
    rb1 = rb0 + 2
    rr = lane & 7

    def coords(k):
        g0 = base + k * CN
        t = g0 >> 14
        cb0 = pl.multiple_of((g0 & (N - 1)) >> 7, CB)
        return g0, t, cb0

    def transpose_chunk(b, tb):
        tp = tile_v.at[tb]

        @plsc.parallel_loop(0, CN, step=8, unroll=2)
        def nblk(n0b):
            n0b = pl.multiple_of(n0b, 8)
            m = lax.shift_right_logical(n0b, 7)
            mv = jnp.full((L,), 0, jnp.int32) + m
            j0 = n0b & 127
            for u in range(8):
                n = n0b + u
                jv = jnp.full((L,), 0, jnp.int32) + (j0 + u)
                v0 = rows_v[b, n, pl.ds(0, L)]
                v1 = rows_v[b, n, pl.ds(L, L)]
                plsc.store_scatter(tp, [rb0, mv, rr, jv], v0)
                plsc.store_scatter(tp, [rb1, mv, rr, jv], v1)

    for b in range(NRB):
        g0, _, _ = coords(b)
        pltpu.sync_copy(idx_hbm.at[pl.ds(g0, CN)], idx_v.at[b])
        pltpu.async_copy(table_hbm.at[idx_v.at[b]], rows_v.at[b], gsems[b])

    def group(g, carry):
        for b in range(NRB):
            k = NRB * g + b
            tb = b % 2
            g0, t, cb0 = coords(k)
            pltpu.make_async_copy(
                table_hbm.at[idx_v.at[b]], rows_v.at[b], gsems[b]
            ).wait()

            @pl.when(g < NGRP - 1)
            def _pfi(b=b, k=k):
                g0n, _, _ = coords(k + NRB)
                pltpu.async_copy(
                    idx_hbm.at[pl.ds(g0n, CN)], idx_v.at[b], isems[b]
                )

            @pl.when(k >= 2)
            def _drain(tb=tb, k=k):
                _, tpp, cb0p = coords(k - 2)
                pltpu.make_async_copy(
                    tile_v.at[tb, :, :, :, pl.ds(0, 128)],
                    out_hbm.at[tpp, :, pl.ds(cb0p, CB)],
                    wsems[tb],
                ).wait()

            transpose_chunk(b, tb)
            pltpu.async_copy(
                tile_v.at[tb, :, :, :, pl.ds(0, 128)],
                out_hbm.at[t, :, pl.ds(cb0, CB)],
                wsems[tb],
            )

            @pl.when(g < NGRP - 1)
            def _pfg(b=b):
                pltpu.make_async_copy(
                    idx_hbm.at[pl.ds(0, CN)], idx_v.at[b], isems[b]
                ).wait()
                pltpu.async_copy(
                    table_hbm.at[idx_v.at[b]], rows_v.at[b], gsems[b]
                )

        return carry

    lax.fori_loop(0, NGRP, group, 0)
    for b in range(2):
        k = NCH - 2 + b
        _, t, cb0 = coords(k)
        pltpu.make_async_copy(
            tile_v.at[k % 2, :, :, :, pl.ds(0, 128)],
            out_hbm.at[t, :, pl.ds(cb0, CB)],
            wsems[k % 2],
        ).wait()


def kernel(round_numbers, embedding_table):
    idx_t = round_numbers.T.reshape(-1)
    out5 = _gather_t(idx_t, embedding_table)
    out_t = out5.transpose(0, 1, 3, 2, 4).reshape(T, D, N)
    return jnp.transpose(out_t, (2, 0, 1))
